# baseline (speedup 1.0000x reference)
import functools
import jax
import jax.numpy as jnp
from jax import lax
from jax.experimental import pallas as pl
from jax.experimental.pallas import tpu as pltpu
from jax.experimental.pallas import tpu_sc as plsc

F2F_THRD = 0.98
F2T_THRD = -1.0
MAX_FRAME_NUM = 32
N = 1000
NPAD = 1024
NCHUNK = NPAD // 16
EMBED = 2560
HID = 512
NP = 16
BF = 40
NB = N // BF
NEG = -1e30


def _fused_tc_kernel(x_ref, t_ref, ltw_ref, ltb_ref, lw_ref, lb_ref,
                     W1_ref, b1_ref, w2_ref, b2_ref,
                     g_ref, gpad_ref, f2f_ref,
                     p_s, tn_s):
    i = pl.program_id(0)

    @pl.when(i == 0)
    def _():
        t = t_ref[...]
        mu = jnp.mean(t, axis=1, keepdims=True)
        d = t - mu
        var = jnp.mean(d * d, axis=1, keepdims=True)
        tn_s[...] = d / jnp.sqrt(var + 1e-5) * ltw_ref[...] + ltb_ref[...]

    pooled = jnp.mean(x_ref[...], axis=1)
    mu = jnp.mean(pooled, axis=1, keepdims=True)
    d = pooled - mu
    var = jnp.mean(d * d, axis=1, keepdims=True)
    p_s[pl.ds(i * BF, BF), :] = (
        d / jnp.sqrt(var + 1e-5) * lw_ref[...] + lb_ref[...])

    @pl.when(i == NB - 1)
    def _():
        p = p_s[...]
        fused = jnp.concatenate(
            [jnp.broadcast_to(tn_s[...].astype(jnp.bfloat16), (N, EMBED)),
             p.astype(jnp.bfloat16)], axis=1)
        h = jnp.dot(fused, W1_ref[...].astype(jnp.bfloat16),
                    preferred_element_type=jnp.float32) + b1_ref[...]
        h = jnp.maximum(h, 0.0)
        logits = jax.lax.dot_general(
            w2_ref[...].astype(jnp.bfloat16), h.astype(jnp.bfloat16),
            (((1,), (1,)), ((), ())),
            preferred_element_type=jnp.float32) + b2_ref[...]
        gates = jax.nn.sigmoid(logits)
        g_ref[...] = gates

        idx = jax.lax.broadcasted_iota(jnp.int32, (1, NPAD), 1)
        gp = jnp.pad(gates, ((0, 0), (0, NPAD - N)))
        gpad_ref[...] = jnp.where(idx < N, gp, NEG)

        nrm = jnp.sqrt(jnp.sum(p * p, axis=1, keepdims=True))
        pn = p / jnp.maximum(nrm, 1e-8)
        f2f_ref[:, :N] = jax.lax.dot_general(
            pn, pn, (((1,), (1,)), ((), ())),
            preferred_element_type=jnp.float32)
        f2f_ref[:, N:] = jnp.zeros((N, NPAD - N), jnp.float32)


def _sc_greedy(gates_hbm, f2f_hbm, sel_hbm, g_v, row16_v, sel_v, tmpf_v, tmpi_v, sem):
    wid = lax.axis_index("s") * 2 + lax.axis_index("c")

    @pl.when(wid == 0)
    def _():
        pltpu.sync_copy(gates_hbm, g_v)
        lane = jax.lax.iota(jnp.int32, 16)
        z16 = jnp.zeros((16,), jnp.int32)
        last = jnp.full((16,), 15, jnp.int32)
        for k in range(NCHUNK):
            sel_v[pl.ds(k * 16, 16)] = z16

        def body(_, cnt):
            bv = g_v[pl.ds(0, 16)]
            bi = lane
            for k in range(1, NCHUNK):
                v = g_v[pl.ds(k * 16, 16)]
                better = v > bv
                bi = jnp.where(better, lane + k * 16, bi)
                bv = jnp.where(better, v, bv)
            m = bv
            for sh in (1, 2, 4, 8):
                idx = jnp.bitwise_and(lane + sh, 15)
                m = jnp.maximum(m, m.at[idx].get(mode="promise_in_bounds"))
            cand = jnp.where(bv == m, bi, NPAD)
            cur = cand
            for sh in (1, 2, 4, 8):
                idx = jnp.bitwise_and(lane + sh, 15)
                cur = jnp.minimum(cur, cur.at[idx].get(mode="promise_in_bounds"))
            active = (m >= F2T_THRD) & (cnt < MAX_FRAME_NUM)
            cur_c = jnp.minimum(cur, N - 1)
            pltpu.async_copy(f2f_hbm.at[cur_c], row16_v, sem).wait()
            for k in range(NCHUNK):
                ids = lane + k * 16
                rv = row16_v[0, pl.ds(k * 16, 16)]
                gv = g_v[pl.ds(k * 16, 16)]
                kill = ((rv > F2F_THRD) | (ids == cur)) & active
                g_v[pl.ds(k * 16, 16)] = jnp.where(kill, NEG, gv)
                slk = sel_v[pl.ds(k * 16, 16)]
                sel_v[pl.ds(k * 16, 16)] = jnp.where(
                    (ids == cur) & active, 1, slk)
            return jnp.where(active, cnt + 1, cnt)

        jax.lax.fori_loop(0, MAX_FRAME_NUM, body,
                          jnp.zeros((16,), jnp.int32))
        pltpu.sync_copy(sel_v, sel_hbm)


def kernel(image_features, text_features, ln_text_w, ln_text_b,
           ln_local_w, ln_local_b, W1, b1, W2, b2):
    gates, gpad, f2f = pl.pallas_call(
        _fused_tc_kernel,
        grid=(NB,),
        in_specs=[
            pl.BlockSpec((BF, NP, EMBED), lambda i: (i, 0, 0)),
            pl.BlockSpec((1, EMBED), lambda i: (0, 0)),
            pl.BlockSpec((1, EMBED), lambda i: (0, 0)),
            pl.BlockSpec((1, EMBED), lambda i: (0, 0)),
            pl.BlockSpec((1, EMBED), lambda i: (0, 0)),
            pl.BlockSpec((1, EMBED), lambda i: (0, 0)),
            pl.BlockSpec((EMBED * 2, HID), lambda i: (0, 0)),
            pl.BlockSpec((1, HID), lambda i: (0, 0)),
            pl.BlockSpec((1, HID), lambda i: (0, 0)),
            pl.BlockSpec((1, 1), lambda i: (0, 0)),
        ],
        out_specs=[
            pl.BlockSpec((1, N), lambda i: (0, 0)),
            pl.BlockSpec((1, NPAD), lambda i: (0, 0)),
            pl.BlockSpec((N, NPAD), lambda i: (0, 0)),
        ],
        out_shape=[
            jax.ShapeDtypeStruct((1, N), jnp.float32),
            jax.ShapeDtypeStruct((1, NPAD), jnp.float32),
            jax.ShapeDtypeStruct((N, NPAD), jnp.float32),
        ],
        scratch_shapes=[
            pltpu.VMEM((N, EMBED), jnp.float32),
            pltpu.VMEM((1, EMBED), jnp.float32),
        ],
    )(image_features, text_features,
      ln_text_w.reshape(1, EMBED), ln_text_b.reshape(1, EMBED),
      ln_local_w.reshape(1, EMBED), ln_local_b.reshape(1, EMBED),
      W1, b1.reshape(1, HID), W2.reshape(1, HID), b2.reshape(1, 1))

    sc_fn = functools.partial(
        pl.kernel,
        out_type=jax.ShapeDtypeStruct((NPAD,), jnp.int32),
        mesh=plsc.VectorSubcoreMesh(core_axis_name="c", subcore_axis_name="s"),
        scratch_types=[
            pltpu.VMEM((NPAD,), jnp.float32),
            pltpu.VMEM((16, NPAD), jnp.float32),
            pltpu.VMEM((NPAD,), jnp.int32),
            pltpu.VMEM((16,), jnp.float32),
            pltpu.VMEM((16,), jnp.int32),
            pltpu.SemaphoreType.DMA,
        ],
    )(_sc_greedy)
    sel = sc_fn(gpad.reshape(NPAD), f2f)

    return (sel[:N], gates[0])


# speedup vs baseline: 1.8896x; 1.8896x over previous
import jax
import jax.numpy as jnp
from jax.experimental import pallas as pl
from jax.experimental.pallas import tpu as pltpu

F2F_THRD = 0.98
F2T_THRD = -1.0
MAX_FRAME_NUM = 32
N = 1000
EMBED = 2560
HID = 512
NP = 16
BF = 40
NB = N // BF


def _fused_kernel(x_ref, t_ref, ltw_ref, ltb_ref, lw_ref, lb_ref,
                  W1_ref, b1_ref, w2_ref, b2_ref,
                  sel_ref, g_ref,
                  p_s, tn_s, f2f_s, vis_s):
    i = pl.program_id(0)

    @pl.when(i == 0)
    def _():
        t = t_ref[...]
        mu = jnp.mean(t, axis=1, keepdims=True)
        d = t - mu
        var = jnp.mean(d * d, axis=1, keepdims=True)
        tn_s[...] = d / jnp.sqrt(var + 1e-5) * ltw_ref[...] + ltb_ref[...]

    pooled = jnp.mean(x_ref[...], axis=1)
    mu = jnp.mean(pooled, axis=1, keepdims=True)
    d = pooled - mu
    var = jnp.mean(d * d, axis=1, keepdims=True)
    p_s[pl.ds(i * BF, BF), :] = (
        d / jnp.sqrt(var + 1e-5) * lw_ref[...] + lb_ref[...])

    @pl.when(i == NB - 1)
    def _():
        p = p_s[...]
        fused = jnp.concatenate(
            [jnp.broadcast_to(tn_s[...].astype(jnp.bfloat16), (N, EMBED)),
             p.astype(jnp.bfloat16)], axis=1)
        h = jnp.dot(fused, W1_ref[...].astype(jnp.bfloat16),
                    preferred_element_type=jnp.float32) + b1_ref[...]
        h = jnp.maximum(h, 0.0)
        logits = jax.lax.dot_general(
            w2_ref[...].astype(jnp.bfloat16), h.astype(jnp.bfloat16),
            (((1,), (1,)), ((), ())),
            preferred_element_type=jnp.float32) + b2_ref[...]
        gates = jax.nn.sigmoid(logits)
        g_ref[...] = gates

        nrm = jnp.sqrt(jnp.sum(p * p, axis=1, keepdims=True))
        pn = p / jnp.maximum(nrm, 1e-8)
        f2f_s[...] = jax.lax.dot_general(
            pn, pn, (((1,), (1,)), ((), ())),
            preferred_element_type=jnp.float32)

        idx = jax.lax.broadcasted_iota(jnp.int32, (1, N), 1)
        vis_s[...] = jnp.zeros((1, N), jnp.float32)
        sel_ref[...] = jnp.zeros((1, N), jnp.int32)

        def body(_, cnt):
            g = g_ref[...]
            v = vis_s[...]
            masked = jnp.where(v > 0.0, -jnp.inf, g)
            m = jnp.max(masked)
            cur = jnp.min(jnp.where(masked == m, idx, N))
            active = (m >= F2T_THRD) & (cnt < MAX_FRAME_NUM)
            row = f2f_s[pl.ds(cur, 1), :]
            nv = jnp.where((row > F2F_THRD) | (idx == cur), 1.0, v)
            vis_s[...] = jnp.where(active, nv, v)
            sel_ref[...] = jnp.where(active & (idx == cur), 1, sel_ref[...])
            return cnt + active.astype(jnp.int32)

        jax.lax.fori_loop(0, MAX_FRAME_NUM, body, jnp.int32(0))


def kernel(image_features, text_features, ln_text_w, ln_text_b,
           ln_local_w, ln_local_b, W1, b1, W2, b2):
    sel, gates = pl.pallas_call(
        _fused_kernel,
        grid=(NB,),
        in_specs=[
            pl.BlockSpec((BF, NP, EMBED), lambda i: (i, 0, 0)),
            pl.BlockSpec((1, EMBED), lambda i: (0, 0)),
            pl.BlockSpec((1, EMBED), lambda i: (0, 0)),
            pl.BlockSpec((1, EMBED), lambda i: (0, 0)),
            pl.BlockSpec((1, EMBED), lambda i: (0, 0)),
            pl.BlockSpec((1, EMBED), lambda i: (0, 0)),
            pl.BlockSpec((EMBED * 2, HID), lambda i: (0, 0)),
            pl.BlockSpec((1, HID), lambda i: (0, 0)),
            pl.BlockSpec((1, HID), lambda i: (0, 0)),
            pl.BlockSpec((1, 1), lambda i: (0, 0)),
        ],
        out_specs=[
            pl.BlockSpec((1, N), lambda i: (0, 0)),
            pl.BlockSpec((1, N), lambda i: (0, 0)),
        ],
        out_shape=[
            jax.ShapeDtypeStruct((1, N), jnp.int32),
            jax.ShapeDtypeStruct((1, N), jnp.float32),
        ],
        scratch_shapes=[
            pltpu.VMEM((N, EMBED), jnp.float32),
            pltpu.VMEM((1, EMBED), jnp.float32),
            pltpu.VMEM((N, N), jnp.float32),
            pltpu.VMEM((1, N), jnp.float32),
        ],
    )(image_features, text_features,
      ln_text_w.reshape(1, EMBED), ln_text_b.reshape(1, EMBED),
      ln_local_w.reshape(1, EMBED), ln_local_b.reshape(1, EMBED),
      W1, b1.reshape(1, HID), W2.reshape(1, HID), b2.reshape(1, 1))

    return (sel[0], gates[0])
